# baseline (speedup 1.0000x reference)
import jax
import jax.numpy as jnp
from jax.experimental import pallas as pl
from jax.experimental.pallas import tpu as pltpu

DIM = 1024
INTER = 1024
E = 8
K = 2
T = 2048
TK = T * K
TILE = 256
LANES = 128
MAX_TILES = 23
PAD_ROWS = MAX_TILES * TILE


def _dot_t(a, b):
    return jax.lax.dot_general(
        a, b, (((1,), (1,)), ((), ())), preferred_element_type=jnp.float32
    )


def _router_kernel(x_ref, gw_ref, gb_ref, out_ref):
    x = x_ref[...]
    logits = _dot_t(x, gw_ref[...]) + gb_ref[0:1, :]
    lane = jax.lax.broadcasted_iota(jnp.int32, logits.shape, 1)
    probs = jnp.where(lane < E, jax.nn.sigmoid(logits), -1.0)
    i1 = jnp.argmax(probs, axis=-1)
    oh1 = lane == i1[:, None]
    m1 = jnp.max(probs, axis=-1, keepdims=True)
    probs2 = jnp.where(oh1, -1.0, probs)
    i2 = jnp.argmax(probs2, axis=-1)
    m2 = jnp.max(probs2, axis=-1, keepdims=True)
    s = m1 + m2 + 1e-8
    w1n = m1 / s
    w2n = m2 / s
    out = (
        jnp.where(lane == 0, i1[:, None].astype(jnp.float32), 0.0)
        + jnp.where(lane == 1, i2[:, None].astype(jnp.float32), 0.0)
        + jnp.where(lane == 2, w1n, 0.0)
        + jnp.where(lane == 3, w2n, 0.0)
    )
    out_ref[...] = out


def _group_kernel(te_ref, xs_ref, w1_ref, w3_ref, w2_ref, o_ref):
    x = xs_ref[...]
    h1 = _dot_t(x, w1_ref[0])
    h3 = _dot_t(x, w3_ref[0])
    h = ((h1 * jax.nn.sigmoid(h1)) * h3).astype(jnp.bfloat16)
    o_ref[...] = _dot_t(h, w2_ref[0])


def _shared_kernel(x_ref, f1_ref, f2_ref, f3_ref, y_ref, o_ref):
    x = x_ref[...]
    h1 = _dot_t(x, f1_ref[...])
    h3 = _dot_t(x, f2_ref[...])
    h = ((h1 * jax.nn.sigmoid(h1)) * h3).astype(jnp.bfloat16)
    o_ref[...] = _dot_t(h, f3_ref[...]) + y_ref[...]


def kernel(x, gate_w, gate_b, w1, w2, w3, fc1, fc2, fc3):
    orig_shape = x.shape
    xt = x.reshape(T, DIM)

    gw_pad = jnp.zeros((LANES, DIM), jnp.float32).at[:E].set(gate_w)
    gb_pad = jnp.zeros((8, LANES), jnp.float32).at[:, :E].set(
        jnp.broadcast_to(gate_b, (8, E))
    )

    rout = pl.pallas_call(
        _router_kernel,
        grid=(T // TILE,),
        in_specs=[
            pl.BlockSpec((TILE, DIM), lambda t: (t, 0)),
            pl.BlockSpec((LANES, DIM), lambda t: (0, 0)),
            pl.BlockSpec((8, LANES), lambda t: (0, 0)),
        ],
        out_specs=pl.BlockSpec((TILE, LANES), lambda t: (t, 0)),
        out_shape=jax.ShapeDtypeStruct((T, LANES), jnp.float32),
    )(xt, gw_pad, gb_pad)

    idx = rout[:, :K].astype(jnp.int32)
    wts = rout[:, K : 2 * K]

    ef = idx.reshape(TK)
    oh = (ef[:, None] == jnp.arange(E)[None, :]).astype(jnp.int32)
    csum = jnp.cumsum(oh, axis=0)
    rank = jnp.take_along_axis(csum - oh, ef[:, None], axis=1)[:, 0]
    counts = csum[-1]
    padded = ((counts + TILE - 1) // TILE) * TILE
    ends = jnp.cumsum(padded)
    pstart = ends - padded
    ppos = pstart[ef] + rank
    tok = jnp.arange(TK, dtype=jnp.int32) // K
    tok_pad = jnp.zeros((PAD_ROWS,), jnp.int32).at[ppos].set(tok)
    tile_expert = jnp.clip(
        jnp.searchsorted(ends, jnp.arange(MAX_TILES) * TILE, side="right"),
        0,
        E - 1,
    ).astype(jnp.int32)

    xtb = xt.astype(jnp.bfloat16)
    xs = xtb[tok_pad]
    w1b = w1.astype(jnp.bfloat16)
    w3b = w3.astype(jnp.bfloat16)
    w2b = w2.astype(jnp.bfloat16)

    eo = pl.pallas_call(
        _group_kernel,
        grid_spec=pltpu.PrefetchScalarGridSpec(
            num_scalar_prefetch=1,
            grid=(MAX_TILES,),
            in_specs=[
                pl.BlockSpec((TILE, DIM), lambda j, te: (j, 0)),
                pl.BlockSpec((1, INTER, DIM), lambda j, te: (te[j], 0, 0)),
                pl.BlockSpec((1, INTER, DIM), lambda j, te: (te[j], 0, 0)),
                pl.BlockSpec((1, DIM, INTER), lambda j, te: (te[j], 0, 0)),
            ],
            out_specs=pl.BlockSpec((TILE, DIM), lambda j, te: (j, 0)),
        ),
        out_shape=jax.ShapeDtypeStruct((PAD_ROWS, DIM), jnp.float32),
    )(tile_expert, xs, w1b, w3b, w2b)

    pos = ppos.reshape(T, K)
    ysum = wts[:, 0:1] * eo[pos[:, 0]] + wts[:, 1:2] * eo[pos[:, 1]]

    out = pl.pallas_call(
        _shared_kernel,
        grid=(T // TILE,),
        in_specs=[
            pl.BlockSpec((TILE, DIM), lambda t: (t, 0)),
            pl.BlockSpec((INTER, DIM), lambda t: (0, 0)),
            pl.BlockSpec((INTER, DIM), lambda t: (0, 0)),
            pl.BlockSpec((DIM, INTER), lambda t: (0, 0)),
            pl.BlockSpec((TILE, DIM), lambda t: (t, 0)),
        ],
        out_specs=pl.BlockSpec((TILE, DIM), lambda t: (t, 0)),
        out_shape=jax.ShapeDtypeStruct((T, DIM), jnp.float32),
    )(
        xtb,
        fc1.astype(jnp.bfloat16),
        fc2.astype(jnp.bfloat16),
        fc3.astype(jnp.bfloat16),
        ysum,
    )

    return out.reshape(orig_shape)


# speedup vs baseline: 1.2476x; 1.2476x over previous
import jax
import jax.numpy as jnp
from jax.experimental import pallas as pl
from jax.experimental.pallas import tpu as pltpu

DIM = 1024
INTER = 1024
E = 8
K = 2
T = 2048
TK = T * K
TILE = 256
LANES = 128
MAX_TILES = 23
PAD_ROWS = MAX_TILES * TILE


def _dot_t(a, b):
    return jax.lax.dot_general(
        a, b, (((1,), (1,)), ((), ())), preferred_element_type=jnp.float32
    )


def _router_kernel(x_ref, gw_ref, gb_ref, out_ref):
    x = x_ref[...]
    logits = _dot_t(x, gw_ref[...]) + gb_ref[0:1, :]
    lane = jax.lax.broadcasted_iota(jnp.int32, logits.shape, 1)
    probs = jnp.where(lane < E, jax.nn.sigmoid(logits), -1.0)
    i1 = jnp.argmax(probs, axis=-1)
    oh1 = lane == i1[:, None]
    m1 = jnp.max(probs, axis=-1, keepdims=True)
    probs2 = jnp.where(oh1, -1.0, probs)
    i2 = jnp.argmax(probs2, axis=-1)
    m2 = jnp.max(probs2, axis=-1, keepdims=True)
    s = m1 + m2 + 1e-8
    w1n = m1 / s
    w2n = m2 / s
    out = (
        jnp.where(lane == 0, i1[:, None].astype(jnp.float32), 0.0)
        + jnp.where(lane == 1, i2[:, None].astype(jnp.float32), 0.0)
        + jnp.where(lane == 2, w1n, 0.0)
        + jnp.where(lane == 3, w2n, 0.0)
    )
    out_ref[...] = out


def _group_kernel(meta_ref, xs_ref, w1_ref, w3_ref, w2_ref, o_ref):
    j = pl.program_id(0)
    n_valid = meta_ref[MAX_TILES]

    @pl.when(j < n_valid)
    def _():
        x = xs_ref[...]
        h1 = _dot_t(x, w1_ref[0])
        h3 = _dot_t(x, w3_ref[0])
        h = (h1 * jax.nn.sigmoid(h1)) * h3
        o_ref[...] = _dot_t(h, w2_ref[0])


def _shared_kernel(x_ref, f1_ref, f2_ref, f3_ref, o_ref):
    x = x_ref[...]
    h1 = _dot_t(x, f1_ref[...])
    h3 = _dot_t(x, f2_ref[...])
    h = (h1 * jax.nn.sigmoid(h1)) * h3
    o_ref[...] = _dot_t(h, f3_ref[...])


def _combine_kernel(z_ref, g0_ref, g1_ref, r_ref, o_ref):
    lane = jax.lax.broadcasted_iota(jnp.int32, (TILE, LANES), 1)
    r = r_ref[...]
    w0 = jnp.sum(jnp.where(lane == 2, r, 0.0), axis=1, keepdims=True)
    w1 = jnp.sum(jnp.where(lane == 3, r, 0.0), axis=1, keepdims=True)
    o_ref[...] = z_ref[...] + w0 * g0_ref[...] + w1 * g1_ref[...]


def kernel(x, gate_w, gate_b, w1, w2, w3, fc1, fc2, fc3):
    orig_shape = x.shape
    xt = x.reshape(T, DIM)

    gw_pad = jnp.zeros((LANES, DIM), jnp.float32).at[:E].set(gate_w)
    gb_pad = jnp.zeros((8, LANES), jnp.float32).at[:, :E].set(
        jnp.broadcast_to(gate_b, (8, E))
    )

    rout = pl.pallas_call(
        _router_kernel,
        grid=(T // TILE,),
        in_specs=[
            pl.BlockSpec((TILE, DIM), lambda t: (t, 0)),
            pl.BlockSpec((LANES, DIM), lambda t: (0, 0)),
            pl.BlockSpec((8, LANES), lambda t: (0, 0)),
        ],
        out_specs=pl.BlockSpec((TILE, LANES), lambda t: (t, 0)),
        out_shape=jax.ShapeDtypeStruct((T, LANES), jnp.float32),
    )(xt, gw_pad, gb_pad)

    idx = rout[:, :K].astype(jnp.int32)

    ef = idx.reshape(TK)
    oh = (ef[:, None] == jnp.arange(E)[None, :]).astype(jnp.int32)
    csum = jnp.cumsum(oh, axis=0)
    rank = jnp.take_along_axis(csum - oh, ef[:, None], axis=1)[:, 0]
    counts = csum[-1]
    padded = ((counts + TILE - 1) // TILE) * TILE
    ends = jnp.cumsum(padded)
    pstart = ends - padded
    ppos = pstart[ef] + rank
    tok = jnp.arange(TK, dtype=jnp.int32) // K
    tok_pad = jnp.zeros((PAD_ROWS,), jnp.int32).at[ppos].set(tok)
    tile_expert = jnp.clip(
        jnp.searchsorted(ends, jnp.arange(MAX_TILES) * TILE, side="right"),
        0,
        E - 1,
    ).astype(jnp.int32)
    n_tiles = (ends[-1] // TILE).astype(jnp.int32)
    meta = jnp.concatenate([tile_expert, n_tiles[None]])

    xs = xt[tok_pad]

    eo = pl.pallas_call(
        _group_kernel,
        grid_spec=pltpu.PrefetchScalarGridSpec(
            num_scalar_prefetch=1,
            grid=(MAX_TILES,),
            in_specs=[
                pl.BlockSpec((TILE, DIM), lambda j, te: (j, 0)),
                pl.BlockSpec((1, INTER, DIM), lambda j, te: (te[j], 0, 0)),
                pl.BlockSpec((1, INTER, DIM), lambda j, te: (te[j], 0, 0)),
                pl.BlockSpec((1, DIM, INTER), lambda j, te: (te[j], 0, 0)),
            ],
            out_specs=pl.BlockSpec((TILE, DIM), lambda j, te: (j, 0)),
        ),
        out_shape=jax.ShapeDtypeStruct((PAD_ROWS, DIM), jnp.float32),
    )(meta, xs, w1, w3, w2)

    z = pl.pallas_call(
        _shared_kernel,
        grid=(T // TILE,),
        in_specs=[
            pl.BlockSpec((TILE, DIM), lambda t: (t, 0)),
            pl.BlockSpec((INTER, DIM), lambda t: (0, 0)),
            pl.BlockSpec((INTER, DIM), lambda t: (0, 0)),
            pl.BlockSpec((DIM, INTER), lambda t: (0, 0)),
        ],
        out_specs=pl.BlockSpec((TILE, DIM), lambda t: (t, 0)),
        out_shape=jax.ShapeDtypeStruct((T, DIM), jnp.float32),
    )(xt, fc1, fc2, fc3)

    pos = ppos.reshape(T, K)
    g0 = eo[pos[:, 0]]
    g1 = eo[pos[:, 1]]

    out = pl.pallas_call(
        _combine_kernel,
        grid=(T // TILE,),
        in_specs=[
            pl.BlockSpec((TILE, DIM), lambda t: (t, 0)),
            pl.BlockSpec((TILE, DIM), lambda t: (t, 0)),
            pl.BlockSpec((TILE, DIM), lambda t: (t, 0)),
            pl.BlockSpec((TILE, LANES), lambda t: (t, 0)),
        ],
        out_specs=pl.BlockSpec((TILE, DIM), lambda t: (t, 0)),
        out_shape=jax.ShapeDtypeStruct((T, DIM), jnp.float32),
    )(z, g0, g1, rout)

    return out.reshape(orig_shape)
